# baseline (speedup 1.0000x reference)
import jax
import jax.numpy as jnp
from jax.experimental import pallas as pl


def _copy_block(in_ref, out_ref):
    out_ref[...] = in_ref[...]


def kernel(x):
    flat = x.reshape(-1, x.shape[-1])
    m, n = flat.shape
    bm = 29256
    out = pl.pallas_call(
        _copy_block,
        grid=(pl.cdiv(m, bm),),
        in_specs=[pl.BlockSpec((bm, n), lambda i: (i, 0))],
        out_specs=pl.BlockSpec((bm, n), lambda i: (i, 0)),
        out_shape=jax.ShapeDtypeStruct((m, n), flat.dtype),
    )(flat)
    return out.reshape(x.shape)


# speedup vs baseline: 1.0154x; 1.0154x over previous
import jax
import jax.numpy as jnp
from jax.experimental import pallas as pl


def _copy_block(in_ref, out_ref):
    out_ref[...] = in_ref[...]


def kernel(x):
    flat = x.reshape(-1, x.shape[-1])
    m, n = flat.shape
    bm = 28672
    out = pl.pallas_call(
        _copy_block,
        grid=(pl.cdiv(m, bm),),
        in_specs=[pl.BlockSpec((bm, n), lambda i: (i, 0))],
        out_specs=pl.BlockSpec((bm, n), lambda i: (i, 0)),
        out_shape=jax.ShapeDtypeStruct((m, n), flat.dtype),
    )(flat)
    return out.reshape(x.shape)
